# baseline (speedup 1.0000x reference)
import jax
import jax.numpy as jnp
from jax.experimental import pallas as pl
from jax.experimental.pallas import tpu as pltpu

_B, _N, _C = 8, 20000, 80
_TOPK = 100
_LANES = 128
_ROWS = 160
_NPAD = _ROWS * _LANES
_PREP_BLK = 2000
_NEG = float("-inf")
_BIG = 2 ** 30


_PREP_R = 16
_PREP_BOX = _PREP_R * _LANES


def _prep_body(conf_ref, scores_ref, classes_ref, smax_ref, cls_ref):
    j = pl.program_id(1)
    s = scores_ref[0].reshape(_PREP_R, _LANES, _C)
    c = classes_ref[0].reshape(_PREP_R, _LANES, _C)
    conf = conf_ref[0]
    rio = jax.lax.broadcasted_iota(jnp.int32, (_PREP_R, _LANES), 0)
    lio = jax.lax.broadcasted_iota(jnp.int32, (_PREP_R, _LANES), 1)
    n = j * _PREP_BOX + rio * _LANES + lio
    m = jnp.max(s, axis=2)
    smax_ref[0] = jnp.where((m >= conf) & (n < _N), m, _NEG)
    cm = jnp.max(c, axis=2, keepdims=True)
    cio = jax.lax.broadcasted_iota(jnp.int32, (_PREP_R, _LANES, _C), 2)
    cidx = jnp.min(jnp.where(c == cm, cio, _BIG), axis=2)
    cls_ref[0] = cidx.astype(jnp.float32)


def _nms_body(iou_ref, s0_ref, boxes_ref, cls_ref,
              obox_ref, oscore_ref, ocls_ref, ovalid_ref,
              *scratch):
    s_refs = scratch[:_B]
    area_refs = scratch[_B:]
    for b in range(_B):
        y1 = boxes_ref[b, 0]
        x1 = boxes_ref[b, 1]
        y2 = boxes_ref[b, 2]
        x2 = boxes_ref[b, 3]
        area_refs[b][:] = jnp.maximum(y2 - y1, 0.0) * jnp.maximum(x2 - x1, 0.0)
        s_refs[b][:] = s0_ref[b]
    iou_t = iou_ref[0]

    rowio = jax.lax.broadcasted_iota(jnp.int32, (_ROWS, _LANES), 0)
    laneio = jax.lax.broadcasted_iota(jnp.int32, (_ROWS, _LANES), 1)
    flat = rowio * _LANES + laneio
    lane1 = jax.lax.broadcasted_iota(jnp.int32, (1, _LANES), 1)

    def body(k, carry):
        km = lane1 == k
        out = []
        for b in range(_B):
            vidx, vscore, vy1, vx1, vy2, vx2, vcls = carry[b]
            s = s_refs[b][:]
            m = jnp.max(s)
            keep = m > _NEG
            fid = jnp.min(jnp.where(s == m, flat, _BIG))
            row = fid // _LANES
            lmask = lane1 == (fid % _LANES)

            def pick(c):
                r = boxes_ref[b, c, pl.ds(row, 1), :]
                return jnp.sum(jnp.where(lmask, r, 0.0))

            by1 = pick(0)
            bx1 = pick(1)
            by2 = pick(2)
            bx2 = pick(3)
            crow = cls_ref[b, pl.ds(row, 1), :]
            bcls = jnp.sum(jnp.where(lmask, crow, 0.0))

            a1 = jnp.maximum(by2 - by1, 0.0) * jnp.maximum(bx2 - bx1, 0.0)
            yy1 = jnp.maximum(by1, boxes_ref[b, 0])
            xx1 = jnp.maximum(bx1, boxes_ref[b, 1])
            yy2 = jnp.minimum(by2, boxes_ref[b, 2])
            xx2 = jnp.minimum(bx2, boxes_ref[b, 3])
            inter = jnp.maximum(yy2 - yy1, 0.0) * jnp.maximum(xx2 - xx1, 0.0)
            union = a1 + area_refs[b][:] - inter
            iou = jnp.where(union > 0.0, inter / union, 0.0)
            supp = iou > iou_t
            s_refs[b][:] = jnp.where(supp | (flat == fid), _NEG, s)

            keepm = km & keep
            vidx = jnp.where(km, jnp.where(keep, fid, -1), vidx)
            vscore = jnp.where(km, jnp.where(keep, m, -1.0), vscore)
            vy1 = jnp.where(keepm, by1, vy1)
            vx1 = jnp.where(keepm, bx1, vx1)
            vy2 = jnp.where(keepm, by2, vy2)
            vx2 = jnp.where(keepm, bx2, vx2)
            vcls = jnp.where(km, jnp.where(keep, bcls, -1.0), vcls)
            out.append((vidx, vscore, vy1, vx1, vy2, vx2, vcls))
        return tuple(out)

    zeros = jnp.zeros((1, _LANES), jnp.float32)
    init_b = (jnp.full((1, _LANES), -1, jnp.int32),
              jnp.full((1, _LANES), -1.0, jnp.float32),
              zeros, zeros, zeros, zeros,
              jnp.full((1, _LANES), -1.0, jnp.float32))
    fin = jax.lax.fori_loop(0, _TOPK, body, tuple(init_b for _ in range(_B)))

    for b in range(_B):
        vidx, vscore, vy1, vx1, vy2, vx2, vcls = fin[b]
        obox_ref[b] = jnp.concatenate([vy1, vx1, vy2, vx2], axis=0)
        oscore_ref[b] = vscore
        ocls_ref[b] = vcls
        nv = jnp.sum(jnp.where((lane1 < _TOPK) & (vidx >= 0), 1, 0))
        ovalid_ref[b] = jnp.broadcast_to(nv, (1, _LANES))


def kernel(boxes, classes, scores, topk_all, iou_thres, conf_thres):
    conf = jnp.asarray(conf_thres, jnp.float32).reshape(1)
    iou = jnp.asarray(iou_thres, jnp.float32).reshape(1)

    s0p, clsp = pl.pallas_call(
        _prep_body,
        grid=(_B, _ROWS // _PREP_R),
        in_specs=[
            pl.BlockSpec(memory_space=pltpu.SMEM),
            pl.BlockSpec((1, _PREP_BOX, _C), lambda b, i: (b, i, 0)),
            pl.BlockSpec((1, _PREP_BOX, _C), lambda b, i: (b, i, 0)),
        ],
        out_specs=[
            pl.BlockSpec((1, _PREP_R, _LANES), lambda b, i: (b, i, 0)),
            pl.BlockSpec((1, _PREP_R, _LANES), lambda b, i: (b, i, 0)),
        ],
        out_shape=[
            jax.ShapeDtypeStruct((_B, _ROWS, _LANES), jnp.float32),
            jax.ShapeDtypeStruct((_B, _ROWS, _LANES), jnp.float32),
        ],
    )(conf, scores, classes)

    pad = _NPAD - _N
    boxesp = jnp.pad(boxes, ((0, 0), (0, pad), (0, 0))
                     ).transpose(0, 2, 1).reshape(_B, 4, _ROWS, _LANES)

    obox, oscore, ocls, ovalid = pl.pallas_call(
        _nms_body,
        grid=(1,),
        in_specs=[
            pl.BlockSpec(memory_space=pltpu.SMEM),
            pl.BlockSpec((_B, _ROWS, _LANES), lambda i: (0, 0, 0)),
            pl.BlockSpec((_B, 4, _ROWS, _LANES), lambda i: (0, 0, 0, 0)),
            pl.BlockSpec((_B, _ROWS, _LANES), lambda i: (0, 0, 0)),
        ],
        out_specs=[
            pl.BlockSpec((_B, 4, _LANES), lambda i: (0, 0, 0)),
            pl.BlockSpec((_B, 1, _LANES), lambda i: (0, 0, 0)),
            pl.BlockSpec((_B, 1, _LANES), lambda i: (0, 0, 0)),
            pl.BlockSpec((_B, 1, _LANES), lambda i: (0, 0, 0)),
        ],
        out_shape=[
            jax.ShapeDtypeStruct((_B, 4, _LANES), jnp.float32),
            jax.ShapeDtypeStruct((_B, 1, _LANES), jnp.float32),
            jax.ShapeDtypeStruct((_B, 1, _LANES), jnp.float32),
            jax.ShapeDtypeStruct((_B, 1, _LANES), jnp.int32),
        ],
        scratch_shapes=[pltpu.VMEM((_ROWS, _LANES), jnp.float32)
                        for _ in range(2 * _B)],
    )(iou, s0p, boxesp, clsp)

    padded_boxes = obox[:, :, :_TOPK].transpose(0, 2, 1)
    padded_scores = oscore[:, 0, :_TOPK]
    padded_classes = ocls[:, 0, :_TOPK]
    valid = jnp.minimum(ovalid[:, 0, 0],
                        jnp.asarray(topk_all).astype(jnp.int32))
    return padded_boxes, padded_scores, padded_classes, valid


# speedup vs baseline: 1.0071x; 1.0071x over previous
import jax
import jax.numpy as jnp
from jax.experimental import pallas as pl
from jax.experimental.pallas import tpu as pltpu

_B, _N, _C = 8, 20000, 80
_TOPK = 100
_LANES = 128
_ROWS = 160
_NPAD = _ROWS * _LANES
_NEG = float("-inf")
_BIG = 2 ** 30

_PREP_R = 16
_PREP_BOX = _PREP_R * _LANES


def _prep_body(conf_ref, scores_ref, classes_ref, smax_ref, cls_ref):
    j = pl.program_id(1)
    s = scores_ref[0].reshape(_PREP_R, _LANES, _C)
    c = classes_ref[0].reshape(_PREP_R, _LANES, _C)
    conf = conf_ref[0]
    rio = jax.lax.broadcasted_iota(jnp.int32, (_PREP_R, _LANES), 0)
    lio = jax.lax.broadcasted_iota(jnp.int32, (_PREP_R, _LANES), 1)
    n = j * _PREP_BOX + rio * _LANES + lio
    m = jnp.max(s, axis=2)
    smax_ref[0] = jnp.where((m >= conf) & (n < _N), m, _NEG)
    cm = jnp.max(c, axis=2, keepdims=True)
    cio = jax.lax.broadcasted_iota(jnp.int32, (_PREP_R, _LANES, _C), 2)
    cidx = jnp.min(jnp.where(c == cm, cio, _BIG), axis=2)
    cls_ref[0] = cidx.astype(jnp.float32)


def _rmin2(x):
    return jnp.min(jnp.min(x, axis=1, keepdims=True), axis=0, keepdims=True)


def _rmax2(x):
    return jnp.max(jnp.max(x, axis=1, keepdims=True), axis=0, keepdims=True)


def _rsum2(x):
    return jnp.sum(jnp.sum(x, axis=1, keepdims=True), axis=0, keepdims=True)


def _nms_body(iou_ref, s0_ref, boxes_ref, cls_ref,
              obox_ref, oscore_ref, ocls_ref, ovalid_ref,
              *scratch):
    s_refs = scratch[:_B]
    area_refs = scratch[_B:]
    for b in range(_B):
        y1 = boxes_ref[b, 0]
        x1 = boxes_ref[b, 1]
        y2 = boxes_ref[b, 2]
        x2 = boxes_ref[b, 3]
        area_refs[b][:] = jnp.maximum(y2 - y1, 0.0) * jnp.maximum(x2 - x1, 0.0)
        s_refs[b][:] = s0_ref[b]
    iou_t = iou_ref[0]

    rowio = jax.lax.broadcasted_iota(jnp.int32, (_ROWS, _LANES), 0)
    laneio = jax.lax.broadcasted_iota(jnp.int32, (_ROWS, _LANES), 1)
    flat = rowio * _LANES + laneio
    lane1 = jax.lax.broadcasted_iota(jnp.int32, (1, _LANES), 1)

    def body(k, carry):
        km = lane1 == k
        out = []
        for b in range(_B):
            vidx, vscore, vy1, vx1, vy2, vx2, vcls = carry[b]
            s = s_refs[b][:]
            m11 = _rmax2(s)
            keep = m11 > _NEG
            eq = s == m11
            fid = _rmin2(jnp.where(eq, flat, _BIG))
            selm = flat == fid

            def pick(x):
                return _rsum2(jnp.where(selm, x, 0.0))

            by1 = pick(boxes_ref[b, 0])
            bx1 = pick(boxes_ref[b, 1])
            by2 = pick(boxes_ref[b, 2])
            bx2 = pick(boxes_ref[b, 3])
            bcls = pick(cls_ref[b])

            a1 = jnp.maximum(by2 - by1, 0.0) * jnp.maximum(bx2 - bx1, 0.0)
            yy1 = jnp.maximum(by1, boxes_ref[b, 0])
            xx1 = jnp.maximum(bx1, boxes_ref[b, 1])
            yy2 = jnp.minimum(by2, boxes_ref[b, 2])
            xx2 = jnp.minimum(bx2, boxes_ref[b, 3])
            inter = jnp.maximum(yy2 - yy1, 0.0) * jnp.maximum(xx2 - xx1, 0.0)
            union = a1 + area_refs[b][:] - inter
            iou = jnp.where(union > 0.0, inter / union, 0.0)
            supp = iou > iou_t
            s_refs[b][:] = jnp.where(supp | selm, _NEG, s)

            keepm = km & keep
            vidx = jnp.where(km, jnp.where(keep, fid, -1), vidx)
            vscore = jnp.where(km, jnp.where(keep, m11, -1.0), vscore)
            vy1 = jnp.where(keepm, by1, vy1)
            vx1 = jnp.where(keepm, bx1, vx1)
            vy2 = jnp.where(keepm, by2, vy2)
            vx2 = jnp.where(keepm, bx2, vx2)
            vcls = jnp.where(km, jnp.where(keep, bcls, -1.0), vcls)
            out.append((vidx, vscore, vy1, vx1, vy2, vx2, vcls))
        return tuple(out)

    zeros = jnp.zeros((1, _LANES), jnp.float32)
    init_b = (jnp.full((1, _LANES), -1, jnp.int32),
              jnp.full((1, _LANES), -1.0, jnp.float32),
              zeros, zeros, zeros, zeros,
              jnp.full((1, _LANES), -1.0, jnp.float32))
    fin = jax.lax.fori_loop(0, _TOPK, body, tuple(init_b for _ in range(_B)))

    for b in range(_B):
        vidx, vscore, vy1, vx1, vy2, vx2, vcls = fin[b]
        obox_ref[b] = jnp.concatenate([vy1, vx1, vy2, vx2], axis=0)
        oscore_ref[b] = vscore
        ocls_ref[b] = vcls
        nv = jnp.sum(jnp.where((lane1 < _TOPK) & (vidx >= 0), 1, 0))
        ovalid_ref[b] = jnp.broadcast_to(nv, (1, _LANES))


def kernel(boxes, classes, scores, topk_all, iou_thres, conf_thres):
    conf = jnp.asarray(conf_thres, jnp.float32).reshape(1)
    iou = jnp.asarray(iou_thres, jnp.float32).reshape(1)

    s0p, clsp = pl.pallas_call(
        _prep_body,
        grid=(_B, _ROWS // _PREP_R),
        in_specs=[
            pl.BlockSpec(memory_space=pltpu.SMEM),
            pl.BlockSpec((1, _PREP_BOX, _C), lambda b, i: (b, i, 0)),
            pl.BlockSpec((1, _PREP_BOX, _C), lambda b, i: (b, i, 0)),
        ],
        out_specs=[
            pl.BlockSpec((1, _PREP_R, _LANES), lambda b, i: (b, i, 0)),
            pl.BlockSpec((1, _PREP_R, _LANES), lambda b, i: (b, i, 0)),
        ],
        out_shape=[
            jax.ShapeDtypeStruct((_B, _ROWS, _LANES), jnp.float32),
            jax.ShapeDtypeStruct((_B, _ROWS, _LANES), jnp.float32),
        ],
    )(conf, scores, classes)

    pad = _NPAD - _N
    boxesp = jnp.pad(boxes, ((0, 0), (0, pad), (0, 0))
                     ).transpose(0, 2, 1).reshape(_B, 4, _ROWS, _LANES)

    obox, oscore, ocls, ovalid = pl.pallas_call(
        _nms_body,
        grid=(1,),
        in_specs=[
            pl.BlockSpec(memory_space=pltpu.SMEM),
            pl.BlockSpec((_B, _ROWS, _LANES), lambda i: (0, 0, 0)),
            pl.BlockSpec((_B, 4, _ROWS, _LANES), lambda i: (0, 0, 0, 0)),
            pl.BlockSpec((_B, _ROWS, _LANES), lambda i: (0, 0, 0)),
        ],
        out_specs=[
            pl.BlockSpec((_B, 4, _LANES), lambda i: (0, 0, 0)),
            pl.BlockSpec((_B, 1, _LANES), lambda i: (0, 0, 0)),
            pl.BlockSpec((_B, 1, _LANES), lambda i: (0, 0, 0)),
            pl.BlockSpec((_B, 1, _LANES), lambda i: (0, 0, 0)),
        ],
        out_shape=[
            jax.ShapeDtypeStruct((_B, 4, _LANES), jnp.float32),
            jax.ShapeDtypeStruct((_B, 1, _LANES), jnp.float32),
            jax.ShapeDtypeStruct((_B, 1, _LANES), jnp.float32),
            jax.ShapeDtypeStruct((_B, 1, _LANES), jnp.int32),
        ],
        scratch_shapes=[pltpu.VMEM((_ROWS, _LANES), jnp.float32)
                        for _ in range(2 * _B)],
    )(iou, s0p, boxesp, clsp)

    padded_boxes = obox[:, :, :_TOPK].transpose(0, 2, 1)
    padded_scores = oscore[:, 0, :_TOPK]
    padded_classes = ocls[:, 0, :_TOPK]
    valid = jnp.minimum(ovalid[:, 0, 0],
                        jnp.asarray(topk_all).astype(jnp.int32))
    return padded_boxes, padded_scores, padded_classes, valid


# speedup vs baseline: 1.5336x; 1.5228x over previous
import jax
import jax.numpy as jnp
from jax.experimental import pallas as pl
from jax.experimental.pallas import tpu as pltpu

_B, _N, _C = 8, 20000, 80
_TOPK = 100
_LANES = 128
_ROWS = 160
_NPAD = _ROWS * _LANES
_NEG = float("-inf")
_BIG = 2 ** 30

_PREP_R = 16
_PREP_BOX = _PREP_R * _LANES


def _prep_body(conf_ref, scores_ref, classes_ref, smax_ref, cls_ref):
    j = pl.program_id(1)
    s = scores_ref[0].reshape(_PREP_R, _LANES, _C)
    c = classes_ref[0].reshape(_PREP_R, _LANES, _C)
    conf = conf_ref[0]
    rio = jax.lax.broadcasted_iota(jnp.int32, (_PREP_R, _LANES), 0)
    lio = jax.lax.broadcasted_iota(jnp.int32, (_PREP_R, _LANES), 1)
    n = j * _PREP_BOX + rio * _LANES + lio
    m = jnp.max(s, axis=2)
    smax_ref[0] = jnp.where((m >= conf) & (n < _N), m, _NEG)
    cm = jnp.max(c, axis=2, keepdims=True)
    cio = jax.lax.broadcasted_iota(jnp.int32, (_PREP_R, _LANES, _C), 2)
    cidx = jnp.min(jnp.where(c == cm, cio, _BIG), axis=2)
    cls_ref[0] = cidx.astype(jnp.float32)


def _rmin2(x):
    return jnp.min(jnp.min(x, axis=0, keepdims=True), axis=1, keepdims=True)


def _rmax2(x):
    return jnp.max(jnp.max(x, axis=0, keepdims=True), axis=1, keepdims=True)


def _rsum2(x):
    return jnp.sum(jnp.sum(x, axis=0, keepdims=True), axis=1, keepdims=True)


def _nms_body(iou_ref, s0_ref, boxes_ref, cls_ref,
              obox_ref, oscore_ref, ocls_ref, ovalid_ref,
              *scratch):
    s_refs = scratch[:_B]
    area_refs = scratch[_B:]
    for b in range(_B):
        y1 = boxes_ref[b, 0]
        x1 = boxes_ref[b, 1]
        y2 = boxes_ref[b, 2]
        x2 = boxes_ref[b, 3]
        area_refs[b][:] = jnp.maximum(y2 - y1, 0.0) * jnp.maximum(x2 - x1, 0.0)
        s_refs[b][:] = s0_ref[b]
    iou_t = iou_ref[0]

    rowio = jax.lax.broadcasted_iota(jnp.int32, (_ROWS, _LANES), 0)
    laneio = jax.lax.broadcasted_iota(jnp.int32, (_ROWS, _LANES), 1)
    flat = rowio * _LANES + laneio
    lane1 = jax.lax.broadcasted_iota(jnp.int32, (1, _LANES), 1)

    def body(k, carry):
        km = lane1 == k
        out = []
        for b in range(_B):
            vidx, vscore, vy1, vx1, vy2, vx2, vcls = carry[b]
            s = s_refs[b][:]
            m11 = _rmax2(s)
            keep = m11 > _NEG
            eq = s == m11
            fid = _rmin2(jnp.where(eq, flat, _BIG))
            selm = flat == fid

            def pick(x):
                return _rsum2(jnp.where(selm, x, 0.0))

            by1 = pick(boxes_ref[b, 0])
            bx1 = pick(boxes_ref[b, 1])
            by2 = pick(boxes_ref[b, 2])
            bx2 = pick(boxes_ref[b, 3])
            bcls = pick(cls_ref[b])

            a1 = jnp.maximum(by2 - by1, 0.0) * jnp.maximum(bx2 - bx1, 0.0)
            yy1 = jnp.maximum(by1, boxes_ref[b, 0])
            xx1 = jnp.maximum(bx1, boxes_ref[b, 1])
            yy2 = jnp.minimum(by2, boxes_ref[b, 2])
            xx2 = jnp.minimum(bx2, boxes_ref[b, 3])
            inter = jnp.maximum(yy2 - yy1, 0.0) * jnp.maximum(xx2 - xx1, 0.0)
            union = a1 + area_refs[b][:] - inter
            iou = jnp.where(union > 0.0, inter / union, 0.0)
            supp = iou > iou_t
            s_refs[b][:] = jnp.where(supp | selm, _NEG, s)

            keepm = km & keep
            vidx = jnp.where(km, jnp.where(keep, fid, -1), vidx)
            vscore = jnp.where(km, jnp.where(keep, m11, -1.0), vscore)
            vy1 = jnp.where(keepm, by1, vy1)
            vx1 = jnp.where(keepm, bx1, vx1)
            vy2 = jnp.where(keepm, by2, vy2)
            vx2 = jnp.where(keepm, bx2, vx2)
            vcls = jnp.where(km, jnp.where(keep, bcls, -1.0), vcls)
            out.append((vidx, vscore, vy1, vx1, vy2, vx2, vcls))
        return tuple(out)

    zeros = jnp.zeros((1, _LANES), jnp.float32)
    init_b = (jnp.full((1, _LANES), -1, jnp.int32),
              jnp.full((1, _LANES), -1.0, jnp.float32),
              zeros, zeros, zeros, zeros,
              jnp.full((1, _LANES), -1.0, jnp.float32))
    fin = jax.lax.fori_loop(0, _TOPK, body, tuple(init_b for _ in range(_B)))

    for b in range(_B):
        vidx, vscore, vy1, vx1, vy2, vx2, vcls = fin[b]
        obox_ref[b] = jnp.concatenate([vy1, vx1, vy2, vx2], axis=0)
        oscore_ref[b] = vscore
        ocls_ref[b] = vcls
        nv = jnp.sum(jnp.where((lane1 < _TOPK) & (vidx >= 0), 1, 0))
        ovalid_ref[b] = jnp.broadcast_to(nv, (1, _LANES))


def kernel(boxes, classes, scores, topk_all, iou_thres, conf_thres):
    conf = jnp.asarray(conf_thres, jnp.float32).reshape(1)
    iou = jnp.asarray(iou_thres, jnp.float32).reshape(1)

    s0p, clsp = pl.pallas_call(
        _prep_body,
        grid=(_B, _ROWS // _PREP_R),
        in_specs=[
            pl.BlockSpec(memory_space=pltpu.SMEM),
            pl.BlockSpec((1, _PREP_BOX, _C), lambda b, i: (b, i, 0)),
            pl.BlockSpec((1, _PREP_BOX, _C), lambda b, i: (b, i, 0)),
        ],
        out_specs=[
            pl.BlockSpec((1, _PREP_R, _LANES), lambda b, i: (b, i, 0)),
            pl.BlockSpec((1, _PREP_R, _LANES), lambda b, i: (b, i, 0)),
        ],
        out_shape=[
            jax.ShapeDtypeStruct((_B, _ROWS, _LANES), jnp.float32),
            jax.ShapeDtypeStruct((_B, _ROWS, _LANES), jnp.float32),
        ],
    )(conf, scores, classes)

    pad = _NPAD - _N
    boxesp = jnp.pad(boxes, ((0, 0), (0, pad), (0, 0))
                     ).transpose(0, 2, 1).reshape(_B, 4, _ROWS, _LANES)

    obox, oscore, ocls, ovalid = pl.pallas_call(
        _nms_body,
        grid=(1,),
        in_specs=[
            pl.BlockSpec(memory_space=pltpu.SMEM),
            pl.BlockSpec((_B, _ROWS, _LANES), lambda i: (0, 0, 0)),
            pl.BlockSpec((_B, 4, _ROWS, _LANES), lambda i: (0, 0, 0, 0)),
            pl.BlockSpec((_B, _ROWS, _LANES), lambda i: (0, 0, 0)),
        ],
        out_specs=[
            pl.BlockSpec((_B, 4, _LANES), lambda i: (0, 0, 0)),
            pl.BlockSpec((_B, 1, _LANES), lambda i: (0, 0, 0)),
            pl.BlockSpec((_B, 1, _LANES), lambda i: (0, 0, 0)),
            pl.BlockSpec((_B, 1, _LANES), lambda i: (0, 0, 0)),
        ],
        out_shape=[
            jax.ShapeDtypeStruct((_B, 4, _LANES), jnp.float32),
            jax.ShapeDtypeStruct((_B, 1, _LANES), jnp.float32),
            jax.ShapeDtypeStruct((_B, 1, _LANES), jnp.float32),
            jax.ShapeDtypeStruct((_B, 1, _LANES), jnp.int32),
        ],
        scratch_shapes=[pltpu.VMEM((_ROWS, _LANES), jnp.float32)
                        for _ in range(2 * _B)],
    )(iou, s0p, boxesp, clsp)

    padded_boxes = obox[:, :, :_TOPK].transpose(0, 2, 1)
    padded_scores = oscore[:, 0, :_TOPK]
    padded_classes = ocls[:, 0, :_TOPK]
    valid = jnp.minimum(ovalid[:, 0, 0],
                        jnp.asarray(topk_all).astype(jnp.int32))
    return padded_boxes, padded_scores, padded_classes, valid


# speedup vs baseline: 1.7501x; 1.1412x over previous
import jax
import jax.numpy as jnp
from jax.experimental import pallas as pl
from jax.experimental.pallas import tpu as pltpu

_B, _N, _C = 8, 20000, 80
_TOPK = 100
_LANES = 128
_ROWS = 160
_NPAD = _ROWS * _LANES
_NEG = float("-inf")
_BIG = 2 ** 30

_PREP_R = 16
_PREP_BOX = _PREP_R * _LANES


def _prep_body(conf_ref, scores_ref, smax_ref):
    j = pl.program_id(1)
    s = scores_ref[0].reshape(_PREP_R, _LANES, _C)
    conf = conf_ref[0]
    rio = jax.lax.broadcasted_iota(jnp.int32, (_PREP_R, _LANES), 0)
    lio = jax.lax.broadcasted_iota(jnp.int32, (_PREP_R, _LANES), 1)
    n = j * _PREP_BOX + rio * _LANES + lio
    m = jnp.max(s, axis=2)
    smax_ref[0] = jnp.where((m >= conf) & (n < _N), m, _NEG)


def _cls_body(oidx_ref, classes_ref, ocls_ref):
    lane1 = jax.lax.broadcasted_iota(jnp.int32, (1, _LANES), 1)
    res = jnp.full((1, _LANES), -1.0, jnp.float32)
    for k in range(_TOPK):
        idx = oidx_ref[0, 0, k]
        safe = jnp.maximum(idx, 0)
        rowv = classes_ref[0, pl.ds(safe, 1), :]
        cm = jnp.max(rowv, axis=1, keepdims=True)
        cio = jax.lax.broadcasted_iota(jnp.int32, (1, _C), 1)
        cidx = jnp.min(jnp.where(rowv == cm, cio, _BIG),
                       axis=1, keepdims=True).astype(jnp.float32)
        val = jnp.where(idx >= 0, cidx, -1.0)
        res = jnp.where(lane1 == k, val, res)
    ocls_ref[0] = res


def _rmin2(x):
    return jnp.min(jnp.min(x, axis=0, keepdims=True), axis=1, keepdims=True)


def _rmax2(x):
    return jnp.max(jnp.max(x, axis=0, keepdims=True), axis=1, keepdims=True)


def _rsum2(x):
    return jnp.sum(jnp.sum(x, axis=0, keepdims=True), axis=1, keepdims=True)


def _nms_body(iou_ref, s0_ref, boxes_ref,
              obox_ref, oscore_ref, oidx_ref, ovalid_ref,
              *scratch):
    s_refs = scratch[:_B]
    area_refs = scratch[_B:]
    for b in range(_B):
        y1 = boxes_ref[b, 0]
        x1 = boxes_ref[b, 1]
        y2 = boxes_ref[b, 2]
        x2 = boxes_ref[b, 3]
        area_refs[b][:] = jnp.maximum(y2 - y1, 0.0) * jnp.maximum(x2 - x1, 0.0)
        s_refs[b][:] = s0_ref[b]
    iou_t = iou_ref[0]

    rowio = jax.lax.broadcasted_iota(jnp.int32, (_ROWS, _LANES), 0)
    laneio = jax.lax.broadcasted_iota(jnp.int32, (_ROWS, _LANES), 1)
    flat = rowio * _LANES + laneio
    lane1 = jax.lax.broadcasted_iota(jnp.int32, (1, _LANES), 1)

    def body(k, carry):
        km = lane1 == k
        out = []
        for b in range(_B):
            vidx, vscore, vy1, vx1, vy2, vx2 = carry[b]
            s = s_refs[b][:]
            m11 = _rmax2(s)
            keep = m11 > _NEG
            eq = s == m11
            fid = _rmin2(jnp.where(eq, flat, _BIG))
            selm = flat == fid

            def pick(x):
                return _rsum2(jnp.where(selm, x, 0.0))

            by1 = pick(boxes_ref[b, 0])
            bx1 = pick(boxes_ref[b, 1])
            by2 = pick(boxes_ref[b, 2])
            bx2 = pick(boxes_ref[b, 3])

            a1 = jnp.maximum(by2 - by1, 0.0) * jnp.maximum(bx2 - bx1, 0.0)
            yy1 = jnp.maximum(by1, boxes_ref[b, 0])
            xx1 = jnp.maximum(bx1, boxes_ref[b, 1])
            yy2 = jnp.minimum(by2, boxes_ref[b, 2])
            xx2 = jnp.minimum(bx2, boxes_ref[b, 3])
            inter = jnp.maximum(yy2 - yy1, 0.0) * jnp.maximum(xx2 - xx1, 0.0)
            union = a1 + area_refs[b][:] - inter
            iou = jnp.where(union > 0.0, inter / union, 0.0)
            supp = iou > iou_t
            s_refs[b][:] = jnp.where(supp | selm, _NEG, s)

            keepm = km & keep
            vidx = jnp.where(km, jnp.where(keep, fid, -1), vidx)
            vscore = jnp.where(km, jnp.where(keep, m11, -1.0), vscore)
            vy1 = jnp.where(keepm, by1, vy1)
            vx1 = jnp.where(keepm, bx1, vx1)
            vy2 = jnp.where(keepm, by2, vy2)
            vx2 = jnp.where(keepm, bx2, vx2)
            out.append((vidx, vscore, vy1, vx1, vy2, vx2))
        return tuple(out)

    zeros = jnp.zeros((1, _LANES), jnp.float32)
    init_b = (jnp.full((1, _LANES), -1, jnp.int32),
              jnp.full((1, _LANES), -1.0, jnp.float32),
              zeros, zeros, zeros, zeros)
    fin = jax.lax.fori_loop(0, _TOPK, body, tuple(init_b for _ in range(_B)))

    for b in range(_B):
        vidx, vscore, vy1, vx1, vy2, vx2 = fin[b]
        obox_ref[b] = jnp.concatenate([vy1, vx1, vy2, vx2], axis=0)
        oscore_ref[b] = vscore
        oidx_ref[b] = vidx
        nv = jnp.sum(jnp.where((lane1 < _TOPK) & (vidx >= 0), 1, 0))
        ovalid_ref[b] = jnp.broadcast_to(nv, (1, _LANES))


def kernel(boxes, classes, scores, topk_all, iou_thres, conf_thres):
    conf = jnp.asarray(conf_thres, jnp.float32).reshape(1)
    iou = jnp.asarray(iou_thres, jnp.float32).reshape(1)

    s0p = pl.pallas_call(
        _prep_body,
        grid=(_B, _ROWS // _PREP_R),
        in_specs=[
            pl.BlockSpec(memory_space=pltpu.SMEM),
            pl.BlockSpec((1, _PREP_BOX, _C), lambda b, i: (b, i, 0)),
        ],
        out_specs=pl.BlockSpec((1, _PREP_R, _LANES), lambda b, i: (b, i, 0)),
        out_shape=jax.ShapeDtypeStruct((_B, _ROWS, _LANES), jnp.float32),
    )(conf, scores)

    pad = _NPAD - _N
    boxesp = jnp.pad(boxes, ((0, 0), (0, pad), (0, 0))
                     ).transpose(0, 2, 1).reshape(_B, 4, _ROWS, _LANES)

    obox, oscore, oidx, ovalid = pl.pallas_call(
        _nms_body,
        grid=(1,),
        in_specs=[
            pl.BlockSpec(memory_space=pltpu.SMEM),
            pl.BlockSpec((_B, _ROWS, _LANES), lambda i: (0, 0, 0)),
            pl.BlockSpec((_B, 4, _ROWS, _LANES), lambda i: (0, 0, 0, 0)),
        ],
        out_specs=[
            pl.BlockSpec((_B, 4, _LANES), lambda i: (0, 0, 0)),
            pl.BlockSpec((_B, 1, _LANES), lambda i: (0, 0, 0)),
            pl.BlockSpec((_B, 1, _LANES), lambda i: (0, 0, 0)),
            pl.BlockSpec((_B, 1, _LANES), lambda i: (0, 0, 0)),
        ],
        out_shape=[
            jax.ShapeDtypeStruct((_B, 4, _LANES), jnp.float32),
            jax.ShapeDtypeStruct((_B, 1, _LANES), jnp.float32),
            jax.ShapeDtypeStruct((_B, 1, _LANES), jnp.int32),
            jax.ShapeDtypeStruct((_B, 1, _LANES), jnp.int32),
        ],
        scratch_shapes=[pltpu.VMEM((_ROWS, _LANES), jnp.float32)
                        for _ in range(2 * _B)],
    )(iou, s0p, boxesp)

    ocls = pl.pallas_call(
        _cls_body,
        grid=(_B,),
        in_specs=[
            pl.BlockSpec((1, 1, _LANES), lambda b: (b, 0, 0),
                         memory_space=pltpu.SMEM),
            pl.BlockSpec((1, _N, _C), lambda b: (b, 0, 0)),
        ],
        out_specs=pl.BlockSpec((1, 1, _LANES), lambda b: (b, 0, 0)),
        out_shape=jax.ShapeDtypeStruct((_B, 1, _LANES), jnp.float32),
    )(oidx, classes)

    padded_boxes = obox[:, :, :_TOPK].transpose(0, 2, 1)
    padded_scores = oscore[:, 0, :_TOPK]
    padded_classes = ocls[:, 0, :_TOPK]
    valid = jnp.minimum(ovalid[:, 0, 0],
                        jnp.asarray(topk_all).astype(jnp.int32))
    return padded_boxes, padded_scores, padded_classes, valid


# speedup vs baseline: 1.9555x; 1.1174x over previous
import jax
import jax.numpy as jnp
from jax.experimental import pallas as pl
from jax.experimental.pallas import tpu as pltpu

_B, _N, _C = 8, 20000, 80
_TOPK = 100
_LANES = 128
_ROWS = 160
_NPAD = _ROWS * _LANES
_NEG = float("-inf")
_BIG = 2 ** 30

_PREP_R = 80
_PREP_BOX = _PREP_R * _LANES


def _prep_body(conf_ref, scores_ref, smax_ref):
    j = pl.program_id(1)
    s = scores_ref[0].reshape(_PREP_R, _LANES, _C)
    conf = conf_ref[0]
    rio = jax.lax.broadcasted_iota(jnp.int32, (_PREP_R, _LANES), 0)
    lio = jax.lax.broadcasted_iota(jnp.int32, (_PREP_R, _LANES), 1)
    n = j * _PREP_BOX + rio * _LANES + lio
    m = jnp.max(s, axis=2)
    smax_ref[0] = jnp.where((m >= conf) & (n < _N), m, _NEG)


def _cls_body(oidx_ref, oidxv_ref, classes_ref, ocls_ref, rows_ref, sem):
    handles = []
    for b in range(_B):
        for k in range(_TOPK):
            safe = jnp.maximum(oidx_ref[b, 0, k], 0)
            handles.append(pltpu.make_async_copy(
                classes_ref.at[b, pl.ds(safe, 1), :],
                rows_ref.at[b, pl.ds(k, 1), :], sem))
    for h in handles:
        h.start()
    for h in handles:
        h.wait()
    r = rows_ref[...]
    cm = jnp.max(r, axis=2, keepdims=True)
    cio = jax.lax.broadcasted_iota(jnp.int32, (_B, _LANES, _C), 2)
    cidx = jnp.min(jnp.where(r == cm, cio, _BIG), axis=2).astype(jnp.float32)
    ocls_ref[...] = jnp.where(oidxv_ref[:, 0, :] >= 0, cidx, -1.0)


def _rmin2(x):
    return jnp.min(jnp.min(x, axis=0, keepdims=True), axis=1, keepdims=True)


def _rmax2(x):
    return jnp.max(jnp.max(x, axis=0, keepdims=True), axis=1, keepdims=True)


def _rsum2(x):
    return jnp.sum(jnp.sum(x, axis=0, keepdims=True), axis=1, keepdims=True)


def _nms_body(iou_ref, s0_ref, boxes_ref,
              obox_ref, oscore_ref, oidx_ref, ovalid_ref,
              *scratch):
    s_refs = scratch[:_B]
    area_refs = scratch[_B:]
    for b in range(_B):
        y1 = boxes_ref[b, 0]
        x1 = boxes_ref[b, 1]
        y2 = boxes_ref[b, 2]
        x2 = boxes_ref[b, 3]
        area_refs[b][:] = jnp.maximum(y2 - y1, 0.0) * jnp.maximum(x2 - x1, 0.0)
        s_refs[b][:] = s0_ref[b]
    iou_t = iou_ref[0]

    rowio = jax.lax.broadcasted_iota(jnp.int32, (_ROWS, _LANES), 0)
    laneio = jax.lax.broadcasted_iota(jnp.int32, (_ROWS, _LANES), 1)
    flat = rowio * _LANES + laneio
    lane1 = jax.lax.broadcasted_iota(jnp.int32, (1, _LANES), 1)

    def body(k, carry):
        km = lane1 == k
        out = []
        for b in range(_B):
            vidx, vscore, vy1, vx1, vy2, vx2 = carry[b]
            s = s_refs[b][:]
            m11 = _rmax2(s)
            keep = m11 > _NEG
            eq = s == m11
            fid = _rmin2(jnp.where(eq, flat, _BIG))
            selm = flat == fid

            def pick(x):
                return _rsum2(jnp.where(selm, x, 0.0))

            by1 = pick(boxes_ref[b, 0])
            bx1 = pick(boxes_ref[b, 1])
            by2 = pick(boxes_ref[b, 2])
            bx2 = pick(boxes_ref[b, 3])

            a1 = jnp.maximum(by2 - by1, 0.0) * jnp.maximum(bx2 - bx1, 0.0)
            yy1 = jnp.maximum(by1, boxes_ref[b, 0])
            xx1 = jnp.maximum(bx1, boxes_ref[b, 1])
            yy2 = jnp.minimum(by2, boxes_ref[b, 2])
            xx2 = jnp.minimum(bx2, boxes_ref[b, 3])
            inter = jnp.maximum(yy2 - yy1, 0.0) * jnp.maximum(xx2 - xx1, 0.0)
            union = a1 + area_refs[b][:] - inter
            iou = jnp.where(union > 0.0, inter / union, 0.0)
            supp = iou > iou_t
            s_refs[b][:] = jnp.where(supp | selm, _NEG, s)

            keepm = km & keep
            vidx = jnp.where(km, jnp.where(keep, fid, -1), vidx)
            vscore = jnp.where(km, jnp.where(keep, m11, -1.0), vscore)
            vy1 = jnp.where(keepm, by1, vy1)
            vx1 = jnp.where(keepm, bx1, vx1)
            vy2 = jnp.where(keepm, by2, vy2)
            vx2 = jnp.where(keepm, bx2, vx2)
            out.append((vidx, vscore, vy1, vx1, vy2, vx2))
        return tuple(out)

    zeros = jnp.zeros((1, _LANES), jnp.float32)
    init_b = (jnp.full((1, _LANES), -1, jnp.int32),
              jnp.full((1, _LANES), -1.0, jnp.float32),
              zeros, zeros, zeros, zeros)
    fin = jax.lax.fori_loop(0, _TOPK, body, tuple(init_b for _ in range(_B)))

    for b in range(_B):
        vidx, vscore, vy1, vx1, vy2, vx2 = fin[b]
        obox_ref[b] = jnp.concatenate([vy1, vx1, vy2, vx2], axis=0)
        oscore_ref[b] = vscore
        oidx_ref[b] = vidx
        nv = jnp.sum(jnp.where((lane1 < _TOPK) & (vidx >= 0), 1, 0))
        ovalid_ref[b] = jnp.broadcast_to(nv, (1, _LANES))


def kernel(boxes, classes, scores, topk_all, iou_thres, conf_thres):
    conf = jnp.asarray(conf_thres, jnp.float32).reshape(1)
    iou = jnp.asarray(iou_thres, jnp.float32).reshape(1)

    s0p = pl.pallas_call(
        _prep_body,
        grid=(_B, _ROWS // _PREP_R),
        in_specs=[
            pl.BlockSpec(memory_space=pltpu.SMEM),
            pl.BlockSpec((1, _PREP_BOX, _C), lambda b, i: (b, i, 0)),
        ],
        out_specs=pl.BlockSpec((1, _PREP_R, _LANES), lambda b, i: (b, i, 0)),
        out_shape=jax.ShapeDtypeStruct((_B, _ROWS, _LANES), jnp.float32),
    )(conf, scores)

    pad = _NPAD - _N
    boxesp = jnp.pad(boxes, ((0, 0), (0, pad), (0, 0))
                     ).transpose(0, 2, 1).reshape(_B, 4, _ROWS, _LANES)

    obox, oscore, oidx, ovalid = pl.pallas_call(
        _nms_body,
        grid=(1,),
        in_specs=[
            pl.BlockSpec(memory_space=pltpu.SMEM),
            pl.BlockSpec((_B, _ROWS, _LANES), lambda i: (0, 0, 0)),
            pl.BlockSpec((_B, 4, _ROWS, _LANES), lambda i: (0, 0, 0, 0)),
        ],
        out_specs=[
            pl.BlockSpec((_B, 4, _LANES), lambda i: (0, 0, 0)),
            pl.BlockSpec((_B, 1, _LANES), lambda i: (0, 0, 0)),
            pl.BlockSpec((_B, 1, _LANES), lambda i: (0, 0, 0)),
            pl.BlockSpec((_B, 1, _LANES), lambda i: (0, 0, 0)),
        ],
        out_shape=[
            jax.ShapeDtypeStruct((_B, 4, _LANES), jnp.float32),
            jax.ShapeDtypeStruct((_B, 1, _LANES), jnp.float32),
            jax.ShapeDtypeStruct((_B, 1, _LANES), jnp.int32),
            jax.ShapeDtypeStruct((_B, 1, _LANES), jnp.int32),
        ],
        scratch_shapes=[pltpu.VMEM((_ROWS, _LANES), jnp.float32)
                        for _ in range(2 * _B)],
    )(iou, s0p, boxesp)

    ocls2 = pl.pallas_call(
        _cls_body,
        grid=(1,),
        in_specs=[
            pl.BlockSpec(memory_space=pltpu.SMEM),
            pl.BlockSpec((_B, 1, _LANES), lambda i: (0, 0, 0)),
            pl.BlockSpec(memory_space=pl.ANY),
        ],
        out_specs=pl.BlockSpec((_B, _LANES), lambda i: (0, 0)),
        out_shape=jax.ShapeDtypeStruct((_B, _LANES), jnp.float32),
        scratch_shapes=[
            pltpu.VMEM((_B, _LANES, _C), jnp.float32),
            pltpu.SemaphoreType.DMA,
        ],
    )(oidx, oidx, classes)
    ocls = ocls2.reshape(_B, 1, _LANES)

    padded_boxes = obox[:, :, :_TOPK].transpose(0, 2, 1)
    padded_scores = oscore[:, 0, :_TOPK]
    padded_classes = ocls[:, 0, :_TOPK]
    valid = jnp.minimum(ovalid[:, 0, 0],
                        jnp.asarray(topk_all).astype(jnp.int32))
    return padded_boxes, padded_scores, padded_classes, valid


# speedup vs baseline: 2.0427x; 1.0446x over previous
import jax
import jax.numpy as jnp
from jax.experimental import pallas as pl
from jax.experimental.pallas import tpu as pltpu

_B, _N, _C = 8, 20000, 80
_TOPK = 100
_LANES = 128
_ROWS = 160
_NPAD = _ROWS * _LANES
_NEG = float("-inf")
_BIG = 2 ** 30

_PREP_R = 80
_PREP_BOX = _PREP_R * _LANES


def _prep_body(conf_ref, scores_ref, smax_ref):
    j = pl.program_id(1)
    s = scores_ref[0].reshape(_PREP_R, _LANES, _C)
    conf = conf_ref[0]
    rio = jax.lax.broadcasted_iota(jnp.int32, (_PREP_R, _LANES), 0)
    lio = jax.lax.broadcasted_iota(jnp.int32, (_PREP_R, _LANES), 1)
    n = j * _PREP_BOX + rio * _LANES + lio
    m = jnp.max(s, axis=2)
    smax_ref[0] = jnp.where((m >= conf) & (n < _N), m, _NEG)


def _cls_body(oidx_ref, oidxv_ref, classes_ref, ocls_ref, rows_ref, sem):
    handles = []
    for b in range(_B):
        for k in range(_TOPK):
            safe = jnp.maximum(oidx_ref[b, 0, k], 0)
            handles.append(pltpu.make_async_copy(
                classes_ref.at[b, pl.ds(safe, 1), :],
                rows_ref.at[b, pl.ds(k, 1), :], sem))
    for h in handles:
        h.start()
    for h in handles:
        h.wait()
    r = rows_ref[...]
    cm = jnp.max(r, axis=2, keepdims=True)
    cio = jax.lax.broadcasted_iota(jnp.int32, (_B, _LANES, _C), 2)
    cidx = jnp.min(jnp.where(r == cm, cio, _BIG), axis=2).astype(jnp.float32)
    ocls_ref[...] = jnp.where(oidxv_ref[:, 0, :] >= 0, cidx, -1.0)


def _rmin2(x):
    return jnp.min(jnp.min(x, axis=0, keepdims=True), axis=1, keepdims=True)


def _rmax2(x):
    return jnp.max(jnp.max(x, axis=0, keepdims=True), axis=1, keepdims=True)


def _rsum2(x):
    return jnp.sum(jnp.sum(x, axis=0, keepdims=True), axis=1, keepdims=True)


def _nms_body(iou_ref, s0_ref, boxes_ref,
              obox_ref, oscore_ref, oidx_ref, ovalid_ref,
              *scratch):
    s_refs = scratch[:_B]
    area_refs = scratch[_B:]
    for b in range(_B):
        y1 = boxes_ref[b, 0]
        x1 = boxes_ref[b, 1]
        y2 = boxes_ref[b, 2]
        x2 = boxes_ref[b, 3]
        area_refs[b][:] = jnp.maximum(y2 - y1, 0.0) * jnp.maximum(x2 - x1, 0.0)
        s_refs[b][:] = s0_ref[b]
    iou_t = iou_ref[0]

    rowio = jax.lax.broadcasted_iota(jnp.int32, (_ROWS, _LANES), 0)
    laneio = jax.lax.broadcasted_iota(jnp.int32, (_ROWS, _LANES), 1)
    flat = rowio * _LANES + laneio
    lane1 = jax.lax.broadcasted_iota(jnp.int32, (1, _LANES), 1)

    def body(k, carry):
        km = lane1 == k
        out = []
        for b in range(_B):
            vidx, vscore, vy1, vx1, vy2, vx2 = carry[b]
            s = s_refs[b][:]
            m11 = _rmax2(s)
            keep = m11 > _NEG
            eq = s == m11
            fid = jnp.min(jnp.where(eq, flat, _BIG))
            selm = flat == fid
            row = fid // _LANES
            lmask = lane1 == (fid % _LANES)

            def pick(c):
                r = boxes_ref[b, c, pl.ds(row, 1), :]
                return jnp.sum(jnp.where(lmask, r, 0.0),
                               axis=1, keepdims=True)

            by1 = pick(0)
            bx1 = pick(1)
            by2 = pick(2)
            bx2 = pick(3)

            a1 = jnp.maximum(by2 - by1, 0.0) * jnp.maximum(bx2 - bx1, 0.0)
            yy1 = jnp.maximum(by1, boxes_ref[b, 0])
            xx1 = jnp.maximum(bx1, boxes_ref[b, 1])
            yy2 = jnp.minimum(by2, boxes_ref[b, 2])
            xx2 = jnp.minimum(bx2, boxes_ref[b, 3])
            inter = jnp.maximum(yy2 - yy1, 0.0) * jnp.maximum(xx2 - xx1, 0.0)
            union = a1 + area_refs[b][:] - inter
            iou = jnp.where(union > 0.0, inter / union, 0.0)
            supp = iou > iou_t
            s_refs[b][:] = jnp.where(supp | selm, _NEG, s)

            keepm = km & keep
            vidx = jnp.where(km, jnp.where(keep, fid, -1), vidx)
            vscore = jnp.where(km, jnp.where(keep, m11, -1.0), vscore)
            vy1 = jnp.where(keepm, by1, vy1)
            vx1 = jnp.where(keepm, bx1, vx1)
            vy2 = jnp.where(keepm, by2, vy2)
            vx2 = jnp.where(keepm, bx2, vx2)
            out.append((vidx, vscore, vy1, vx1, vy2, vx2))
        return tuple(out)

    zeros = jnp.zeros((1, _LANES), jnp.float32)
    init_b = (jnp.full((1, _LANES), -1, jnp.int32),
              jnp.full((1, _LANES), -1.0, jnp.float32),
              zeros, zeros, zeros, zeros)
    fin = jax.lax.fori_loop(0, _TOPK, body, tuple(init_b for _ in range(_B)))

    for b in range(_B):
        vidx, vscore, vy1, vx1, vy2, vx2 = fin[b]
        obox_ref[b] = jnp.concatenate([vy1, vx1, vy2, vx2], axis=0)
        oscore_ref[b] = vscore
        oidx_ref[b] = vidx
        nv = jnp.sum(jnp.where((lane1 < _TOPK) & (vidx >= 0), 1, 0))
        ovalid_ref[b] = jnp.broadcast_to(nv, (1, _LANES))


def kernel(boxes, classes, scores, topk_all, iou_thres, conf_thres):
    conf = jnp.asarray(conf_thres, jnp.float32).reshape(1)
    iou = jnp.asarray(iou_thres, jnp.float32).reshape(1)

    s0p = pl.pallas_call(
        _prep_body,
        grid=(_B, _ROWS // _PREP_R),
        in_specs=[
            pl.BlockSpec(memory_space=pltpu.SMEM),
            pl.BlockSpec((1, _PREP_BOX, _C), lambda b, i: (b, i, 0)),
        ],
        out_specs=pl.BlockSpec((1, _PREP_R, _LANES), lambda b, i: (b, i, 0)),
        out_shape=jax.ShapeDtypeStruct((_B, _ROWS, _LANES), jnp.float32),
    )(conf, scores)

    pad = _NPAD - _N
    boxesp = jnp.pad(boxes, ((0, 0), (0, pad), (0, 0))
                     ).transpose(0, 2, 1).reshape(_B, 4, _ROWS, _LANES)

    obox, oscore, oidx, ovalid = pl.pallas_call(
        _nms_body,
        grid=(1,),
        in_specs=[
            pl.BlockSpec(memory_space=pltpu.SMEM),
            pl.BlockSpec((_B, _ROWS, _LANES), lambda i: (0, 0, 0)),
            pl.BlockSpec((_B, 4, _ROWS, _LANES), lambda i: (0, 0, 0, 0)),
        ],
        out_specs=[
            pl.BlockSpec((_B, 4, _LANES), lambda i: (0, 0, 0)),
            pl.BlockSpec((_B, 1, _LANES), lambda i: (0, 0, 0)),
            pl.BlockSpec((_B, 1, _LANES), lambda i: (0, 0, 0)),
            pl.BlockSpec((_B, 1, _LANES), lambda i: (0, 0, 0)),
        ],
        out_shape=[
            jax.ShapeDtypeStruct((_B, 4, _LANES), jnp.float32),
            jax.ShapeDtypeStruct((_B, 1, _LANES), jnp.float32),
            jax.ShapeDtypeStruct((_B, 1, _LANES), jnp.int32),
            jax.ShapeDtypeStruct((_B, 1, _LANES), jnp.int32),
        ],
        scratch_shapes=[pltpu.VMEM((_ROWS, _LANES), jnp.float32)
                        for _ in range(2 * _B)],
    )(iou, s0p, boxesp)

    ocls2 = pl.pallas_call(
        _cls_body,
        grid=(1,),
        in_specs=[
            pl.BlockSpec(memory_space=pltpu.SMEM),
            pl.BlockSpec((_B, 1, _LANES), lambda i: (0, 0, 0)),
            pl.BlockSpec(memory_space=pl.ANY),
        ],
        out_specs=pl.BlockSpec((_B, _LANES), lambda i: (0, 0)),
        out_shape=jax.ShapeDtypeStruct((_B, _LANES), jnp.float32),
        scratch_shapes=[
            pltpu.VMEM((_B, _LANES, _C), jnp.float32),
            pltpu.SemaphoreType.DMA,
        ],
    )(oidx, oidx, classes)
    ocls = ocls2.reshape(_B, 1, _LANES)

    padded_boxes = obox[:, :, :_TOPK].transpose(0, 2, 1)
    padded_scores = oscore[:, 0, :_TOPK]
    padded_classes = ocls[:, 0, :_TOPK]
    valid = jnp.minimum(ovalid[:, 0, 0],
                        jnp.asarray(topk_all).astype(jnp.int32))
    return padded_boxes, padded_scores, padded_classes, valid
